# Initial kernel scaffold; baseline (speedup 1.0000x reference)
import jax
import jax.numpy as jnp
from jax.experimental import pallas as pl


def kernel(x, edge_index, enc_w1, enc_b1, ln_g, ln_b, enc_w2, enc_b2, gat_W, att_src, att_dst, gat_b, skip_W, skip_b, dec_w1, dec_b1, dec_w2, dec_b2):
    raise NotImplementedError("write your pallas kernel here")


# speedup vs baseline: 41.5605x; 41.5605x over previous
import functools

import jax
import jax.numpy as jnp
from jax import lax
from jax.experimental import pallas as pl
from jax.experimental.pallas import tpu as pltpu
from jax.experimental.pallas import tpu_sc as plsc

NC = 2
NS = 16
LL = 16

F32 = jnp.float32
I32 = jnp.int32


def _stage1_body(x_ref, w1_ref, b1_ref, lg_ref, lb_ref, w2_ref, b2_ref,
                 gw_ref, atts_ref, attd_ref, skw_ref, skb_ref,
                 xh_ref, asrc_ref, adst_ref, sself_ref, skip_ref):
    x = x_ref[...]
    h = jnp.dot(x, w1_ref[...], preferred_element_type=F32) + b1_ref[...]
    mu = jnp.mean(h, axis=-1, keepdims=True)
    var = jnp.mean((h - mu) ** 2, axis=-1, keepdims=True)
    h = (h - mu) * lax.rsqrt(var + 1e-5) * lg_ref[...] + lb_ref[...]
    h = jnp.maximum(h, 0.0)
    h = jnp.dot(h, w2_ref[...], preferred_element_type=F32) + b2_ref[...]
    xh = jnp.dot(h, gw_ref[...], preferred_element_type=F32)
    hc = xh.shape[1]
    c = hc // 2
    ps = xh * atts_ref[...]
    pd = xh * attd_ref[...]
    a_s = jnp.stack([jnp.sum(ps[:, :c], axis=1), jnp.sum(ps[:, c:], axis=1)],
                    axis=1)
    a_d = jnp.stack([jnp.sum(pd[:, :c], axis=1), jnp.sum(pd[:, c:], axis=1)],
                    axis=1)
    e_self = a_s + a_d
    e_self = jnp.where(e_self > 0, e_self, 0.2 * e_self)
    sself_ref[...] = jnp.exp(e_self)
    asrc_ref[...] = a_s
    adst_ref[...] = a_d
    skip_ref[...] = jnp.dot(h, skw_ref[...], preferred_element_type=F32) \
        + skb_ref[...]
    for p in range(4):
        xh_ref[p] = xh[:, p * 128:(p + 1) * 128]


def _stage1(x, enc_w1, enc_b1, ln_g, ln_b, enc_w2, enc_b2, gat_W,
            att_src, att_dst, skip_W, skip_b):
    n, d_in = x.shape
    hc = gat_W.shape[1]
    bn = 1000
    grid = (n // bn,)
    full = lambda *shape: pl.BlockSpec(shape, lambda i: (0,) * len(shape))
    row = lambda *shape: pl.BlockSpec(shape, lambda i: (i,) + (0,) * (len(shape) - 1))
    return pl.pallas_call(
        _stage1_body,
        grid=grid,
        in_specs=[
            row(bn, d_in),
            full(d_in, 128), full(1, 128), full(1, 128), full(1, 128),
            full(128, 128), full(1, 128),
            full(128, hc), full(1, hc), full(1, hc),
            full(128, hc), full(1, hc),
        ],
        out_specs=[
            pl.BlockSpec((4, bn, 128), lambda i: (0, i, 0)),
            row(bn, 2), row(bn, 2), row(bn, 2),
            row(bn, hc),
        ],
        out_shape=[
            jax.ShapeDtypeStruct((4, n, 128), F32),
            jax.ShapeDtypeStruct((n, 2), F32),
            jax.ShapeDtypeStruct((n, 2), F32),
            jax.ShapeDtypeStruct((n, 2), F32),
            jax.ShapeDtypeStruct((n, hc), F32),
        ],
    )(x, enc_w1, enc_b1.reshape(1, -1), ln_g.reshape(1, -1),
      ln_b.reshape(1, -1), enc_w2, enc_b2.reshape(1, -1), gat_W,
      att_src.reshape(1, -1), att_dst.reshape(1, -1), skip_W,
      skip_b.reshape(1, -1))


def _splat(vec, lane):
    idx = jnp.full((LL, 1), lane, I32)
    dn = lax.GatherDimensionNumbers(offset_dims=(), collapsed_slice_dims=(0,),
                                    start_index_map=(0,))
    return lax.gather(vec, idx, dn, slice_sizes=(1,),
                      mode=lax.GatherScatterMode.PROMISE_IN_BOUNDS)


def _sc_body(n, nacc, nden, nwin, xh_ref, asrc_ref, adst_ref, srcm_ref,
             dstm_ref, acc_out, den_out,
             src_i, dst_i, id2_v, as_v, ad_v, s_v, gbuf, zbuf, sem_g, sem_s,
             acc_sp, s_sp, den_sp, aS_sp, aD_sp):
    c = lax.axis_index("c")
    t = lax.axis_index("s")
    rpt = 624
    tail0 = rpt * NS
    nw_t = (nwin + NS - 1 - t) // NS

    def _zero_zbuf(i, _):
        zbuf[pl.ds(i * LL, LL)] = jnp.zeros((LL,), F32)
        return 0
    lax.fori_loop(0, zbuf.shape[0] // LL, _zero_zbuf, 0)

    @pl.when(t == 0)
    def _():
        pltpu.sync_copy(asrc_ref, aS_sp)
        pltpu.sync_copy(adst_ref, aD_sp)
        nz = zbuf.shape[0]
        for k in range(nden // nz):
            pltpu.sync_copy(zbuf, den_sp.at[pl.ds(k * nz, nz)])
    plsc.subcore_barrier()

    cvec = jnp.full((LL,), c, I32)

    def _phase_a(i, _):
        w = t + i * NS
        r0 = w * 8
        pltpu.sync_copy(srcm_ref.at[pl.ds(r0, 8)], src_i)
        pltpu.sync_copy(dstm_ref.at[pl.ds(r0, 8)], dst_i)

        def adj(g, _):
            j = g // 8
            k = g % 8
            src_i[j, pl.ds(k * LL, LL)] = \
                src_i[j, pl.ds(k * LL, LL)] * 2 + cvec
            id2_v[j, pl.ds(k * LL, LL)] = \
                dst_i[j, pl.ds(k * LL, LL)] * 2 + cvec
            return 0
        lax.fori_loop(0, 64, adj, 0)

        gds = [pltpu.async_copy(aS_sp.at[src_i.at[j]], as_v.at[j], sem_g)
               for j in range(8)]
        gds += [pltpu.async_copy(aD_sp.at[id2_v.at[j]], ad_v.at[j], sem_g)
                for j in range(8)]
        for d in gds:
            d.wait()

        def grp(g, _):
            j = g // 8
            k = g % 8
            e = as_v[j, pl.ds(k * LL, LL)] + ad_v[j, pl.ds(k * LL, LL)]
            e = jnp.where(e > 0, e, 0.2 * e)
            s_v[j, pl.ds(k * LL, LL)] = jnp.exp(e)
            return 0
        lax.fori_loop(0, 64, grp, 0)

        pltpu.sync_copy(s_v, s_sp.at[c, pl.ds(r0, 8)])
        descs = [pltpu.async_copy(s_v.at[j], den_sp.at[dst_i.at[j]], sem_s,
                                  add=True) for j in range(8)]
        for d in descs:
            d.wait()
        return 0
    lax.fori_loop(0, nw_t, _phase_a, 0)
    plsc.subcore_barrier()

    @pl.when(t == 0)
    def _():
        pltpu.sync_copy(den_sp, den_out.at[c, 0])

    for q in range(2):
        def _zero_g(i, _):
            j = i // 8
            k = i % 8
            gbuf[j, pl.ds(k * LL, LL)] = jnp.zeros((LL,), F32)
            return 0
        lax.fori_loop(0, 256 * 8, _zero_g, 0)
        pltpu.sync_copy(gbuf, acc_sp.at[pl.ds(t * rpt, 256)])
        pltpu.sync_copy(gbuf, acc_sp.at[pl.ds(t * rpt + 256, 256)])
        pltpu.sync_copy(gbuf.at[pl.ds(0, rpt - 512)],
                        acc_sp.at[pl.ds(t * rpt + 512, rpt - 512)])

        @pl.when(t == NS - 1)
        def _():
            pltpu.sync_copy(gbuf.at[pl.ds(0, nacc - tail0)],
                            acc_sp.at[pl.ds(tail0, nacc - tail0)])
        plsc.subcore_barrier()

        off = (2 * c + q) * n

        def _phase_b(i, _):
            w = t + i * NS
            r0 = w * 8
            pltpu.sync_copy(srcm_ref.at[pl.ds(r0, 8)], src_i)
            pltpu.sync_copy(dstm_ref.at[pl.ds(r0, 8)], dst_i)
            pltpu.sync_copy(s_sp.at[c, pl.ds(r0, 8)], s_v)

            def adj(g, _):
                j = g // 8
                k = g % 8
                src_i[j, pl.ds(k * LL, LL)] = \
                    src_i[j, pl.ds(k * LL, LL)] + off
                return 0
            lax.fori_loop(0, 64, adj, 0)

            for quarter in range(4):
                gds = [pltpu.async_copy(xh_ref.at[src_i.at[2 * quarter + j]],
                                        gbuf.at[pl.ds(j * 128, 128)], sem_g)
                       for j in range(2)]
                for d in gds:
                    d.wait()

                def mgrp(g, _):
                    svec = s_v[2 * quarter + g // 8, pl.ds((g % 8) * LL, LL)]
                    e0 = g * LL
                    for jl in range(LL):
                        sj = _splat(svec, jl)
                        ei = e0 + jl
                        for m in range(8):
                            gbuf[ei, pl.ds(m * LL, LL)] = \
                                gbuf[ei, pl.ds(m * LL, LL)] * sj
                    return 0
                lax.fori_loop(0, 16, mgrp, 0)

                sds = [pltpu.async_copy(gbuf.at[pl.ds(j * 128, 128)],
                                        acc_sp.at[dst_i.at[2 * quarter + j]],
                                        sem_s, add=True)
                       for j in range(2)]
                for d in sds:
                    d.wait()
            return 0
        lax.fori_loop(0, nw_t, _phase_b, 0)
        plsc.subcore_barrier()
        pltpu.sync_copy(acc_sp.at[pl.ds(t * rpt, rpt)],
                        acc_out.at[c, q, pl.ds(t * rpt, rpt)])

        @pl.when(t == NS - 1)
        def _():
            pltpu.sync_copy(acc_sp.at[pl.ds(tail0, n - tail0)],
                            acc_out.at[c, q, pl.ds(tail0, n - tail0)])
        plsc.subcore_barrier()


def _stage2(xh_flat, a_src, a_dst, srcm, dstm):
    n2 = a_src.shape[0] // 2
    nwin = srcm.shape[0] // 8
    nacc = n2 + 16
    nden = n2 + 2288
    mesh = plsc.VectorSubcoreMesh(core_axis_name="c", subcore_axis_name="s")
    fn = pl.kernel(
        functools.partial(_sc_body, n2, nacc, nden, nwin),
        out_type=(jax.ShapeDtypeStruct((2, 2, n2, 128), F32),
                  jax.ShapeDtypeStruct((2, 1, nden), F32)),
        mesh=mesh,
        scratch_types=[
            pltpu.VMEM((8, 128), I32),
            pltpu.VMEM((8, 128), I32),
            pltpu.VMEM((8, 128), I32),
            pltpu.VMEM((8, 128), F32),
            pltpu.VMEM((8, 128), F32),
            pltpu.VMEM((8, 128), F32),
            pltpu.VMEM((256, 128), F32),
            pltpu.VMEM((1024,), F32),
            pltpu.SemaphoreType.DMA,
            pltpu.SemaphoreType.DMA,
            pltpu.VMEM_SHARED((nacc, 128), F32),
            pltpu.HBM((2, srcm.shape[0], 128), F32),
            pltpu.VMEM_SHARED((nden,), F32),
            pltpu.VMEM_SHARED((2 * n2,), F32),
            pltpu.VMEM_SHARED((2 * n2,), F32),
        ],
    )
    return fn(xh_flat, a_src, a_dst, srcm, dstm)


def _stage3_body(acc_ref, xh_ref, den_ref, ss_ref, skip_ref, gb_ref,
                 dw1_ref, db1_ref, dw2_ref, db2_ref, out_ref):
    ss = ss_ref[...]
    den = den_ref[...] + ss + 1e-16
    parts = []
    for p in range(4):
        h = p // 2
        num = acc_ref[p] + ss[:, h:h + 1] * xh_ref[p]
        parts.append(num / den[:, h:h + 1])
    conv = jnp.concatenate(parts, axis=1) + gb_ref[...]
    hm = conv + skip_ref[...]
    hm = jnp.where(hm > 0, hm, 0.1 * (jnp.exp(hm) - 1.0))
    d1 = jnp.dot(hm, dw1_ref[...], preferred_element_type=F32) + db1_ref[...]
    d1 = jnp.where(d1 > 0, d1, 0.1 * d1)
    out_ref[...] = jnp.dot(d1, dw2_ref[...], preferred_element_type=F32) \
        + db2_ref[...]


def _stage3(acc, xh_stack, den_t, s_self, skip, gat_b, dec_w1, dec_b1,
            dec_w2, dec_b2):
    n = s_self.shape[0]
    hc = skip.shape[1]
    bn = 1000
    grid = (n // bn,)
    full = lambda *shape: pl.BlockSpec(shape, lambda i: (0,) * len(shape))
    row = lambda *shape: pl.BlockSpec(shape, lambda i: (i,) + (0,) * (len(shape) - 1))
    return pl.pallas_call(
        _stage3_body,
        grid=grid,
        in_specs=[
            pl.BlockSpec((4, bn, 128), lambda i: (0, i, 0)),
            pl.BlockSpec((4, bn, 128), lambda i: (0, i, 0)),
            row(bn, 2), row(bn, 2), row(bn, hc),
            full(1, hc),
            full(hc, 256), full(1, 256), full(256, 128), full(1, 128),
        ],
        out_specs=row(bn, 128),
        out_shape=jax.ShapeDtypeStruct((n, 128), F32),
    )(acc, xh_stack, den_t, s_self, skip, gat_b.reshape(1, -1),
      dec_w1, dec_b1.reshape(1, -1), dec_w2, dec_b2.reshape(1, -1))


def kernel(x, edge_index, enc_w1, enc_b1, ln_g, ln_b, enc_w2, enc_b2,
           gat_W, att_src, att_dst, gat_b, skip_W, skip_b,
           dec_w1, dec_b1, dec_w2, dec_b2):
    n = x.shape[0]
    e = edge_index.shape[1]
    xh_stack, a_src, a_dst, s_self, skip = _stage1(
        x, enc_w1, enc_b1, ln_g, ln_b, enc_w2, enc_b2, gat_W,
        att_src, att_dst, skip_W, skip_b)
    xh_flat = xh_stack.reshape(4 * n, 128)
    epad = (-e) % 1024
    src_pad = jnp.zeros((epad,), I32)
    dst_pad = n + (jnp.arange(epad, dtype=I32) % 16)
    srcm = jnp.concatenate([edge_index[0], src_pad]).reshape(-1, 128)
    dstm = jnp.concatenate([edge_index[1], dst_pad]).reshape(-1, 128)
    acc, den = _stage2(xh_flat, a_src.reshape(-1), a_dst.reshape(-1),
                       srcm, dstm)
    den_t = den[:, 0, :n].T
    return _stage3(acc.reshape(4, n, 128), xh_stack, den_t, s_self, skip,
                   gat_b, dec_w1, dec_b1, dec_w2, dec_b2)


# speedup vs baseline: 52.4381x; 1.2617x over previous
import functools

import jax
import jax.numpy as jnp
from jax import lax
from jax.experimental import pallas as pl
from jax.experimental.pallas import tpu as pltpu
from jax.experimental.pallas import tpu_sc as plsc

NC = 2
NS = 16
LL = 16

F32 = jnp.float32
I32 = jnp.int32


def _stage1_body(x_ref, w1_ref, b1_ref, lg_ref, lb_ref, w2_ref, b2_ref,
                 gw_ref, atts_ref, attd_ref, skw_ref, skb_ref,
                 xh_ref, asrc_ref, adst_ref, sself_ref, skip_ref):
    x = x_ref[...]
    h = jnp.dot(x, w1_ref[...], preferred_element_type=F32) + b1_ref[...]
    mu = jnp.mean(h, axis=-1, keepdims=True)
    var = jnp.mean((h - mu) ** 2, axis=-1, keepdims=True)
    h = (h - mu) * lax.rsqrt(var + 1e-5) * lg_ref[...] + lb_ref[...]
    h = jnp.maximum(h, 0.0)
    h = jnp.dot(h, w2_ref[...], preferred_element_type=F32) + b2_ref[...]
    xh = jnp.dot(h, gw_ref[...], preferred_element_type=F32)
    hc = xh.shape[1]
    c = hc // 2
    ps = xh * atts_ref[...]
    pd = xh * attd_ref[...]
    a_s = jnp.stack([jnp.sum(ps[:, :c], axis=1), jnp.sum(ps[:, c:], axis=1)],
                    axis=1)
    a_d = jnp.stack([jnp.sum(pd[:, :c], axis=1), jnp.sum(pd[:, c:], axis=1)],
                    axis=1)
    e_self = a_s + a_d
    e_self = jnp.where(e_self > 0, e_self, 0.2 * e_self)
    sself_ref[...] = jnp.exp(e_self)
    asrc_ref[...] = a_s
    adst_ref[...] = a_d
    skip_ref[...] = jnp.dot(h, skw_ref[...], preferred_element_type=F32) \
        + skb_ref[...]
    for p in range(4):
        xh_ref[p] = xh[:, p * 128:(p + 1) * 128]


def _stage1(x, enc_w1, enc_b1, ln_g, ln_b, enc_w2, enc_b2, gat_W,
            att_src, att_dst, skip_W, skip_b):
    n, d_in = x.shape
    hc = gat_W.shape[1]
    bn = 1000
    grid = (n // bn,)
    full = lambda *shape: pl.BlockSpec(shape, lambda i: (0,) * len(shape))
    row = lambda *shape: pl.BlockSpec(shape, lambda i: (i,) + (0,) * (len(shape) - 1))
    return pl.pallas_call(
        _stage1_body,
        grid=grid,
        in_specs=[
            row(bn, d_in),
            full(d_in, 128), full(1, 128), full(1, 128), full(1, 128),
            full(128, 128), full(1, 128),
            full(128, hc), full(1, hc), full(1, hc),
            full(128, hc), full(1, hc),
        ],
        out_specs=[
            pl.BlockSpec((4, bn, 128), lambda i: (0, i, 0)),
            row(bn, 2), row(bn, 2), row(bn, 2),
            row(bn, hc),
        ],
        out_shape=[
            jax.ShapeDtypeStruct((4, n, 128), F32),
            jax.ShapeDtypeStruct((n, 2), F32),
            jax.ShapeDtypeStruct((n, 2), F32),
            jax.ShapeDtypeStruct((n, 2), F32),
            jax.ShapeDtypeStruct((n, hc), F32),
        ],
    )(x, enc_w1, enc_b1.reshape(1, -1), ln_g.reshape(1, -1),
      ln_b.reshape(1, -1), enc_w2, enc_b2.reshape(1, -1), gat_W,
      att_src.reshape(1, -1), att_dst.reshape(1, -1), skip_W,
      skip_b.reshape(1, -1))


def _splat(vec, lane):
    idx = jnp.full((LL, 1), lane, I32)
    dn = lax.GatherDimensionNumbers(offset_dims=(), collapsed_slice_dims=(0,),
                                    start_index_map=(0,))
    return lax.gather(vec, idx, dn, slice_sizes=(1,),
                      mode=lax.GatherScatterMode.PROMISE_IN_BOUNDS)


def _sc_body(n, nacc, nden, nwin, xh_ref, asrc_ref, adst_ref, srcm_ref,
             dstm_ref, acc_out, den_out,
             src_i, dst_i, id2_v, as_v, ad_v, s_v, gbuf, zbuf, sem_g, sem_s,
             acc_sp, s_sp, den_sp, aS_sp, aD_sp):
    c = lax.axis_index("c")
    t = lax.axis_index("s")
    rpt = 624
    tail0 = rpt * NS
    nw_t = (nwin + NS - 1 - t) // NS

    def _zero_zbuf(i, _):
        zbuf[pl.ds(i * LL, LL)] = jnp.zeros((LL,), F32)
        return 0
    lax.fori_loop(0, zbuf.shape[0] // LL, _zero_zbuf, 0)

    @pl.when(t == 0)
    def _():
        pltpu.sync_copy(asrc_ref, aS_sp)
        pltpu.sync_copy(adst_ref, aD_sp)
        nz = zbuf.shape[0]
        for k in range(nden // nz):
            pltpu.sync_copy(zbuf, den_sp.at[pl.ds(k * nz, nz)])
    plsc.subcore_barrier()

    cvec = jnp.full((LL,), c, I32)

    def _phase_a(i, _):
        w = t + i * NS
        r0 = w * 8
        pltpu.sync_copy(srcm_ref.at[pl.ds(r0, 8)], src_i)
        pltpu.sync_copy(dstm_ref.at[pl.ds(r0, 8)], dst_i)

        def adj(g, _):
            j = g // 8
            k = g % 8
            src_i[j, pl.ds(k * LL, LL)] = \
                src_i[j, pl.ds(k * LL, LL)] * 2 + cvec
            id2_v[j, pl.ds(k * LL, LL)] = \
                dst_i[j, pl.ds(k * LL, LL)] * 2 + cvec
            return 0
        lax.fori_loop(0, 64, adj, 0)

        gds = [pltpu.async_copy(aS_sp.at[src_i.at[j]], as_v.at[j], sem_g)
               for j in range(8)]
        gds += [pltpu.async_copy(aD_sp.at[id2_v.at[j]], ad_v.at[j], sem_g)
                for j in range(8)]
        for d in gds:
            d.wait()

        def grp(g, _):
            j = g // 8
            k = g % 8
            e = as_v[j, pl.ds(k * LL, LL)] + ad_v[j, pl.ds(k * LL, LL)]
            e = jnp.where(e > 0, e, 0.2 * e)
            s_v[j, pl.ds(k * LL, LL)] = jnp.exp(e)
            return 0
        lax.fori_loop(0, 64, grp, 0)

        pltpu.sync_copy(s_v, s_sp.at[c, pl.ds(r0, 8)])
        descs = [pltpu.async_copy(s_v.at[j], den_sp.at[dst_i.at[j]], sem_s,
                                  add=True) for j in range(8)]
        for d in descs:
            d.wait()
        return 0
    lax.fori_loop(0, nw_t, _phase_a, 0)
    plsc.subcore_barrier()

    @pl.when(t == 0)
    def _():
        pltpu.sync_copy(den_sp, den_out.at[c, 0])

    for q in range(2):
        def _zero_g(i, _):
            j = i // 8
            k = i % 8
            gbuf[j, pl.ds(k * LL, LL)] = jnp.zeros((LL,), F32)
            return 0
        lax.fori_loop(0, 256 * 8, _zero_g, 0)
        pltpu.sync_copy(gbuf, acc_sp.at[pl.ds(t * rpt, 256)])
        pltpu.sync_copy(gbuf, acc_sp.at[pl.ds(t * rpt + 256, 256)])
        pltpu.sync_copy(gbuf.at[pl.ds(0, rpt - 512)],
                        acc_sp.at[pl.ds(t * rpt + 512, rpt - 512)])

        @pl.when(t == NS - 1)
        def _():
            pltpu.sync_copy(gbuf.at[pl.ds(0, nacc - tail0)],
                            acc_sp.at[pl.ds(tail0, nacc - tail0)])
        plsc.subcore_barrier()

        off = (2 * c + q) * n

        def _phase_b(i, _):
            w = t + i * NS
            r0 = w * 8
            pltpu.sync_copy(srcm_ref.at[pl.ds(r0, 8)], src_i)
            pltpu.sync_copy(dstm_ref.at[pl.ds(r0, 8)], dst_i)
            pltpu.sync_copy(s_sp.at[c, pl.ds(r0, 8)], s_v)

            def adj(g, _):
                j = g // 8
                k = g % 8
                src_i[j, pl.ds(k * LL, LL)] = \
                    src_i[j, pl.ds(k * LL, LL)] + off
                return 0
            lax.fori_loop(0, 64, adj, 0)

            def _gath(k):
                b = (k % 2) * 128
                return pltpu.async_copy(xh_ref.at[src_i.at[k]],
                                        gbuf.at[pl.ds(b, 128)], sem_g)

            def _mul(k):
                b = (k % 2) * 128

                def mgrp(g, _):
                    svec = s_v[k, pl.ds(g * LL, LL)]
                    e0 = b + g * LL
                    for jl in range(LL):
                        sj = _splat(svec, jl)
                        ei = e0 + jl
                        for m in range(8):
                            gbuf[ei, pl.ds(m * LL, LL)] = \
                                gbuf[ei, pl.ds(m * LL, LL)] * sj
                    return 0
                lax.fori_loop(0, 8, mgrp, 0)

            def _scat(k):
                b = (k % 2) * 128
                return pltpu.async_copy(gbuf.at[pl.ds(b, 128)],
                                        acc_sp.at[dst_i.at[k]], sem_s,
                                        add=True)

            gds = [None] * 8
            sds = [None] * 8
            gds[0] = _gath(0)
            for k in range(8):
                if k + 1 < 8:
                    if k - 1 >= 0:
                        sds[k - 1].wait()
                    gds[k + 1] = _gath(k + 1)
                gds[k].wait()
                _mul(k)
                sds[k] = _scat(k)
            sds[6].wait()
            sds[7].wait()
            return 0
        lax.fori_loop(0, nw_t, _phase_b, 0)
        plsc.subcore_barrier()
        pltpu.sync_copy(acc_sp.at[pl.ds(t * rpt, rpt)],
                        acc_out.at[c, q, pl.ds(t * rpt, rpt)])

        @pl.when(t == NS - 1)
        def _():
            pltpu.sync_copy(acc_sp.at[pl.ds(tail0, n - tail0)],
                            acc_out.at[c, q, pl.ds(tail0, n - tail0)])
        plsc.subcore_barrier()


def _stage2(xh_flat, a_src, a_dst, srcm, dstm):
    n2 = a_src.shape[0] // 2
    nwin = srcm.shape[0] // 8
    nacc = n2 + 16
    nden = n2 + 2288
    mesh = plsc.VectorSubcoreMesh(core_axis_name="c", subcore_axis_name="s")
    fn = pl.kernel(
        functools.partial(_sc_body, n2, nacc, nden, nwin),
        out_type=(jax.ShapeDtypeStruct((2, 2, n2, 128), F32),
                  jax.ShapeDtypeStruct((2, 1, nden), F32)),
        mesh=mesh,
        scratch_types=[
            pltpu.VMEM((8, 128), I32),
            pltpu.VMEM((8, 128), I32),
            pltpu.VMEM((8, 128), I32),
            pltpu.VMEM((8, 128), F32),
            pltpu.VMEM((8, 128), F32),
            pltpu.VMEM((8, 128), F32),
            pltpu.VMEM((256, 128), F32),
            pltpu.VMEM((1024,), F32),
            pltpu.SemaphoreType.DMA,
            pltpu.SemaphoreType.DMA,
            pltpu.VMEM_SHARED((nacc, 128), F32),
            pltpu.HBM((2, srcm.shape[0], 128), F32),
            pltpu.VMEM_SHARED((nden,), F32),
            pltpu.VMEM_SHARED((2 * n2,), F32),
            pltpu.VMEM_SHARED((2 * n2,), F32),
        ],
    )
    return fn(xh_flat, a_src, a_dst, srcm, dstm)


def _stage3_body(acc_ref, xh_ref, den_ref, ss_ref, skip_ref, gb_ref,
                 dw1_ref, db1_ref, dw2_ref, db2_ref, out_ref):
    ss = ss_ref[...]
    den = den_ref[...] + ss + 1e-16
    parts = []
    for p in range(4):
        h = p // 2
        num = acc_ref[p] + ss[:, h:h + 1] * xh_ref[p]
        parts.append(num / den[:, h:h + 1])
    conv = jnp.concatenate(parts, axis=1) + gb_ref[...]
    hm = conv + skip_ref[...]
    hm = jnp.where(hm > 0, hm, 0.1 * (jnp.exp(hm) - 1.0))
    d1 = jnp.dot(hm, dw1_ref[...], preferred_element_type=F32) + db1_ref[...]
    d1 = jnp.where(d1 > 0, d1, 0.1 * d1)
    out_ref[...] = jnp.dot(d1, dw2_ref[...], preferred_element_type=F32) \
        + db2_ref[...]


def _stage3(acc, xh_stack, den_t, s_self, skip, gat_b, dec_w1, dec_b1,
            dec_w2, dec_b2):
    n = s_self.shape[0]
    hc = skip.shape[1]
    bn = 1000
    grid = (n // bn,)
    full = lambda *shape: pl.BlockSpec(shape, lambda i: (0,) * len(shape))
    row = lambda *shape: pl.BlockSpec(shape, lambda i: (i,) + (0,) * (len(shape) - 1))
    return pl.pallas_call(
        _stage3_body,
        grid=grid,
        in_specs=[
            pl.BlockSpec((4, bn, 128), lambda i: (0, i, 0)),
            pl.BlockSpec((4, bn, 128), lambda i: (0, i, 0)),
            row(bn, 2), row(bn, 2), row(bn, hc),
            full(1, hc),
            full(hc, 256), full(1, 256), full(256, 128), full(1, 128),
        ],
        out_specs=row(bn, 128),
        out_shape=jax.ShapeDtypeStruct((n, 128), F32),
    )(acc, xh_stack, den_t, s_self, skip, gat_b.reshape(1, -1),
      dec_w1, dec_b1.reshape(1, -1), dec_w2, dec_b2.reshape(1, -1))


def kernel(x, edge_index, enc_w1, enc_b1, ln_g, ln_b, enc_w2, enc_b2,
           gat_W, att_src, att_dst, gat_b, skip_W, skip_b,
           dec_w1, dec_b1, dec_w2, dec_b2):
    n = x.shape[0]
    e = edge_index.shape[1]
    xh_stack, a_src, a_dst, s_self, skip = _stage1(
        x, enc_w1, enc_b1, ln_g, ln_b, enc_w2, enc_b2, gat_W,
        att_src, att_dst, skip_W, skip_b)
    xh_flat = xh_stack.reshape(4 * n, 128)
    epad = (-e) % 1024
    src_pad = jnp.zeros((epad,), I32)
    dst_pad = n + (jnp.arange(epad, dtype=I32) % 16)
    srcm = jnp.concatenate([edge_index[0], src_pad]).reshape(-1, 128)
    dstm = jnp.concatenate([edge_index[1], dst_pad]).reshape(-1, 128)
    acc, den = _stage2(xh_flat, a_src.reshape(-1), a_dst.reshape(-1),
                       srcm, dstm)
    den_t = den[:, 0, :n].T
    return _stage3(acc.reshape(4, n, 128), xh_stack, den_t, s_self, skip,
                   gat_b, dec_w1, dec_b1, dec_w2, dec_b2)


# speedup vs baseline: 56.4980x; 1.0774x over previous
import functools

import jax
import jax.numpy as jnp
from jax import lax
from jax.experimental import pallas as pl
from jax.experimental.pallas import tpu as pltpu
from jax.experimental.pallas import tpu_sc as plsc

NC = 2
NS = 16
LL = 16

F32 = jnp.float32
I32 = jnp.int32


def _stage1_body(x_ref, w1_ref, b1_ref, lg_ref, lb_ref, w2_ref, b2_ref,
                 gw_ref, atts_ref, attd_ref, skw_ref, skb_ref,
                 xh_ref, asrc_ref, adst_ref, sself_ref, skip_ref):
    x = x_ref[...]
    h = jnp.dot(x, w1_ref[...], preferred_element_type=F32) + b1_ref[...]
    mu = jnp.mean(h, axis=-1, keepdims=True)
    var = jnp.mean((h - mu) ** 2, axis=-1, keepdims=True)
    h = (h - mu) * lax.rsqrt(var + 1e-5) * lg_ref[...] + lb_ref[...]
    h = jnp.maximum(h, 0.0)
    h = jnp.dot(h, w2_ref[...], preferred_element_type=F32) + b2_ref[...]
    xh = jnp.dot(h, gw_ref[...], preferred_element_type=F32)
    hc = xh.shape[1]
    c = hc // 2
    ps = xh * atts_ref[...]
    pd = xh * attd_ref[...]
    a_s = jnp.stack([jnp.sum(ps[:, :c], axis=1), jnp.sum(ps[:, c:], axis=1)],
                    axis=1)
    a_d = jnp.stack([jnp.sum(pd[:, :c], axis=1), jnp.sum(pd[:, c:], axis=1)],
                    axis=1)
    e_self = a_s + a_d
    e_self = jnp.where(e_self > 0, e_self, 0.2 * e_self)
    sself_ref[...] = jnp.exp(e_self)
    asrc_ref[...] = a_s
    adst_ref[...] = a_d
    skip_ref[...] = jnp.dot(h, skw_ref[...], preferred_element_type=F32) \
        + skb_ref[...]
    for p in range(4):
        xh_ref[p] = xh[:, p * 128:(p + 1) * 128]


def _stage1(x, enc_w1, enc_b1, ln_g, ln_b, enc_w2, enc_b2, gat_W,
            att_src, att_dst, skip_W, skip_b):
    n, d_in = x.shape
    hc = gat_W.shape[1]
    bn = 1000
    grid = (n // bn,)
    full = lambda *shape: pl.BlockSpec(shape, lambda i: (0,) * len(shape))
    row = lambda *shape: pl.BlockSpec(shape, lambda i: (i,) + (0,) * (len(shape) - 1))
    return pl.pallas_call(
        _stage1_body,
        grid=grid,
        in_specs=[
            row(bn, d_in),
            full(d_in, 128), full(1, 128), full(1, 128), full(1, 128),
            full(128, 128), full(1, 128),
            full(128, hc), full(1, hc), full(1, hc),
            full(128, hc), full(1, hc),
        ],
        out_specs=[
            pl.BlockSpec((4, bn, 128), lambda i: (0, i, 0)),
            row(bn, 2), row(bn, 2), row(bn, 2),
            row(bn, hc),
        ],
        out_shape=[
            jax.ShapeDtypeStruct((4, n, 128), F32),
            jax.ShapeDtypeStruct((n, 2), F32),
            jax.ShapeDtypeStruct((n, 2), F32),
            jax.ShapeDtypeStruct((n, 2), F32),
            jax.ShapeDtypeStruct((n, hc), F32),
        ],
    )(x, enc_w1, enc_b1.reshape(1, -1), ln_g.reshape(1, -1),
      ln_b.reshape(1, -1), enc_w2, enc_b2.reshape(1, -1), gat_W,
      att_src.reshape(1, -1), att_dst.reshape(1, -1), skip_W,
      skip_b.reshape(1, -1))


def _splat(vec, lane):
    idx = jnp.full((LL, 1), lane, I32)
    dn = lax.GatherDimensionNumbers(offset_dims=(), collapsed_slice_dims=(0,),
                                    start_index_map=(0,))
    return lax.gather(vec, idx, dn, slice_sizes=(1,),
                      mode=lax.GatherScatterMode.PROMISE_IN_BOUNDS)


def _sc_body(n, nacc, nden, nwin, xh_ref, asrc_ref, adst_ref, srcm_ref,
             dstm_ref, acc_out, den_out,
             src_i, dst_i, is2_v, id2_v, as_v, ad_v, s_v, gbuf, zbuf,
             sem_g, sem_s, sem_d, acc_sp, s_sp, den_sp, aS_sp, aD_sp):
    c = lax.axis_index("c")
    t = lax.axis_index("s")
    rpt = 624
    tail0 = rpt * NS
    nw_t = (nwin + NS - 1 - t) // NS

    def _zero_zbuf(i, _):
        zbuf[pl.ds(i * LL, LL)] = jnp.zeros((LL,), F32)
        return 0
    lax.fori_loop(0, zbuf.shape[0] // LL, _zero_zbuf, 0)

    @pl.when(t == 0)
    def _():
        pltpu.sync_copy(asrc_ref, aS_sp)
        pltpu.sync_copy(adst_ref, aD_sp)
        nz = zbuf.shape[0]
        for k in range(nden // nz):
            pltpu.sync_copy(zbuf, den_sp.at[pl.ds(k * nz, nz)])
    plsc.subcore_barrier()

    cvec = jnp.full((LL,), c, I32)

    for q in range(2):
        def _zero_g(i, _):
            j = i // 8
            k = i % 8
            gbuf[j, pl.ds(k * LL, LL)] = jnp.zeros((LL,), F32)
            return 0
        lax.fori_loop(0, 256 * 8, _zero_g, 0)
        pltpu.sync_copy(gbuf, acc_sp.at[pl.ds(t * rpt, 256)])
        pltpu.sync_copy(gbuf, acc_sp.at[pl.ds(t * rpt + 256, 256)])
        pltpu.sync_copy(gbuf.at[pl.ds(0, rpt - 512)],
                        acc_sp.at[pl.ds(t * rpt + 512, rpt - 512)])

        @pl.when(t == NS - 1)
        def _():
            pltpu.sync_copy(gbuf.at[pl.ds(0, nacc - tail0)],
                            acc_sp.at[pl.ds(tail0, nacc - tail0)])
        plsc.subcore_barrier()

        off = (2 * c + q) * n

        def _phase_b(i, _):
            w = t + i * NS
            r0 = w * 8
            ld = [pltpu.async_copy(srcm_ref.at[pl.ds(r0, 8)], src_i, sem_g),
                  pltpu.async_copy(dstm_ref.at[pl.ds(r0, 8)], dst_i, sem_g)]
            if q == 1:
                ld.append(pltpu.async_copy(s_sp.at[c, pl.ds(r0, 8)], s_v,
                                           sem_g))
            for d in ld:
                d.wait()

            if q == 0:
                def adj0(g, _):
                    j = g // 8
                    k = g % 8
                    sv = src_i[j, pl.ds(k * LL, LL)]
                    dv = dst_i[j, pl.ds(k * LL, LL)]
                    is2_v[j, pl.ds(k * LL, LL)] = sv * 2 + cvec
                    id2_v[j, pl.ds(k * LL, LL)] = dv * 2 + cvec
                    src_i[j, pl.ds(k * LL, LL)] = sv + off
                    return 0
                lax.fori_loop(0, 64, adj0, 0)
                ads = [pltpu.async_copy(aS_sp.at[is2_v.at[j]], as_v.at[j],
                                        sem_g) for j in range(8)]
                ads += [pltpu.async_copy(aD_sp.at[id2_v.at[j]], ad_v.at[j],
                                         sem_g) for j in range(8)]
                for d in ads:
                    d.wait()

                def grp(g, _):
                    j = g // 8
                    k = g % 8
                    e = as_v[j, pl.ds(k * LL, LL)] + ad_v[j, pl.ds(k * LL, LL)]
                    e = jnp.where(e > 0, e, 0.2 * e)
                    s_v[j, pl.ds(k * LL, LL)] = jnp.exp(e)
                    return 0
                lax.fori_loop(0, 64, grp, 0)
                pltpu.sync_copy(s_v, s_sp.at[c, pl.ds(r0, 8)])
                dds = [pltpu.async_copy(s_v.at[j], den_sp.at[dst_i.at[j]],
                                        sem_d, add=True) for j in range(8)]
            else:
                def adj(g, _):
                    j = g // 8
                    k = g % 8
                    src_i[j, pl.ds(k * LL, LL)] = \
                        src_i[j, pl.ds(k * LL, LL)] + off
                    return 0
                lax.fori_loop(0, 64, adj, 0)
                dds = []

            def _gath(k):
                b = (k % 2) * 128
                return pltpu.async_copy(xh_ref.at[src_i.at[k]],
                                        gbuf.at[pl.ds(b, 128)], sem_g)

            def _mul(k):
                b = (k % 2) * 128

                def mgrp(g, _):
                    svec = s_v[k, pl.ds(g * LL, LL)]
                    e0 = b + g * LL
                    for jl in range(LL):
                        sj = _splat(svec, jl)
                        ei = e0 + jl
                        for m in range(8):
                            gbuf[ei, pl.ds(m * LL, LL)] = \
                                gbuf[ei, pl.ds(m * LL, LL)] * sj
                    return 0
                lax.fori_loop(0, 8, mgrp, 0)

            def _scat(k):
                b = (k % 2) * 128
                return pltpu.async_copy(gbuf.at[pl.ds(b, 128)],
                                        acc_sp.at[dst_i.at[k]], sem_s,
                                        add=True)

            gds = [None] * 8
            sds = [None] * 8
            gds[0] = _gath(0)
            for k in range(8):
                if k + 1 < 8:
                    if k - 1 >= 0:
                        sds[k - 1].wait()
                    gds[k + 1] = _gath(k + 1)
                gds[k].wait()
                _mul(k)
                sds[k] = _scat(k)
            sds[6].wait()
            sds[7].wait()
            for d in dds:
                d.wait()
            return 0
        lax.fori_loop(0, nw_t, _phase_b, 0)
        plsc.subcore_barrier()
        if q == 0:
            @pl.when(t == 0)
            def _():
                pltpu.sync_copy(den_sp, den_out.at[c, 0])
        pltpu.sync_copy(acc_sp.at[pl.ds(t * rpt, rpt)],
                        acc_out.at[c, q, pl.ds(t * rpt, rpt)])

        @pl.when(t == NS - 1)
        def _():
            pltpu.sync_copy(acc_sp.at[pl.ds(tail0, n - tail0)],
                            acc_out.at[c, q, pl.ds(tail0, n - tail0)])
        plsc.subcore_barrier()


def _stage2(xh_flat, a_src, a_dst, srcm, dstm):
    n2 = a_src.shape[0] // 2
    nwin = srcm.shape[0] // 8
    nacc = n2 + 16
    nden = n2 + 2288
    mesh = plsc.VectorSubcoreMesh(core_axis_name="c", subcore_axis_name="s")
    fn = pl.kernel(
        functools.partial(_sc_body, n2, nacc, nden, nwin),
        out_type=(jax.ShapeDtypeStruct((2, 2, n2, 128), F32),
                  jax.ShapeDtypeStruct((2, 1, nden), F32)),
        mesh=mesh,
        scratch_types=[
            pltpu.VMEM((8, 128), I32),
            pltpu.VMEM((8, 128), I32),
            pltpu.VMEM((8, 128), I32),
            pltpu.VMEM((8, 128), I32),
            pltpu.VMEM((8, 128), F32),
            pltpu.VMEM((8, 128), F32),
            pltpu.VMEM((8, 128), F32),
            pltpu.VMEM((256, 128), F32),
            pltpu.VMEM((1024,), F32),
            pltpu.SemaphoreType.DMA,
            pltpu.SemaphoreType.DMA,
            pltpu.SemaphoreType.DMA,
            pltpu.VMEM_SHARED((nacc, 128), F32),
            pltpu.HBM((2, srcm.shape[0], 128), F32),
            pltpu.VMEM_SHARED((nden,), F32),
            pltpu.VMEM_SHARED((2 * n2,), F32),
            pltpu.VMEM_SHARED((2 * n2,), F32),
        ],
    )
    return fn(xh_flat, a_src, a_dst, srcm, dstm)


def _stage3_body(acc_ref, xh_ref, den_ref, ss_ref, skip_ref, gb_ref,
                 dw1_ref, db1_ref, dw2_ref, db2_ref, out_ref):
    ss = ss_ref[...]
    den = den_ref[...] + ss + 1e-16
    parts = []
    for p in range(4):
        h = p // 2
        num = acc_ref[p] + ss[:, h:h + 1] * xh_ref[p]
        parts.append(num / den[:, h:h + 1])
    conv = jnp.concatenate(parts, axis=1) + gb_ref[...]
    hm = conv + skip_ref[...]
    hm = jnp.where(hm > 0, hm, 0.1 * (jnp.exp(hm) - 1.0))
    d1 = jnp.dot(hm, dw1_ref[...], preferred_element_type=F32) + db1_ref[...]
    d1 = jnp.where(d1 > 0, d1, 0.1 * d1)
    out_ref[...] = jnp.dot(d1, dw2_ref[...], preferred_element_type=F32) \
        + db2_ref[...]


def _stage3(acc, xh_stack, den_t, s_self, skip, gat_b, dec_w1, dec_b1,
            dec_w2, dec_b2):
    n = s_self.shape[0]
    hc = skip.shape[1]
    bn = 1000
    grid = (n // bn,)
    full = lambda *shape: pl.BlockSpec(shape, lambda i: (0,) * len(shape))
    row = lambda *shape: pl.BlockSpec(shape, lambda i: (i,) + (0,) * (len(shape) - 1))
    return pl.pallas_call(
        _stage3_body,
        grid=grid,
        in_specs=[
            pl.BlockSpec((4, bn, 128), lambda i: (0, i, 0)),
            pl.BlockSpec((4, bn, 128), lambda i: (0, i, 0)),
            row(bn, 2), row(bn, 2), row(bn, hc),
            full(1, hc),
            full(hc, 256), full(1, 256), full(256, 128), full(1, 128),
        ],
        out_specs=row(bn, 128),
        out_shape=jax.ShapeDtypeStruct((n, 128), F32),
    )(acc, xh_stack, den_t, s_self, skip, gat_b.reshape(1, -1),
      dec_w1, dec_b1.reshape(1, -1), dec_w2, dec_b2.reshape(1, -1))


def kernel(x, edge_index, enc_w1, enc_b1, ln_g, ln_b, enc_w2, enc_b2,
           gat_W, att_src, att_dst, gat_b, skip_W, skip_b,
           dec_w1, dec_b1, dec_w2, dec_b2):
    n = x.shape[0]
    e = edge_index.shape[1]
    xh_stack, a_src, a_dst, s_self, skip = _stage1(
        x, enc_w1, enc_b1, ln_g, ln_b, enc_w2, enc_b2, gat_W,
        att_src, att_dst, skip_W, skip_b)
    xh_flat = xh_stack.reshape(4 * n, 128)
    epad = (-e) % 1024
    src_pad = jnp.zeros((epad,), I32)
    dst_pad = n + (jnp.arange(epad, dtype=I32) % 16)
    srcm = jnp.concatenate([edge_index[0], src_pad]).reshape(-1, 128)
    dstm = jnp.concatenate([edge_index[1], dst_pad]).reshape(-1, 128)
    acc, den = _stage2(xh_flat, a_src.reshape(-1), a_dst.reshape(-1),
                       srcm, dstm)
    den_t = den[:, 0, :n].T
    return _stage3(acc.reshape(4, n, 128), xh_stack, den_t, s_self, skip,
                   gat_b, dec_w1, dec_b1, dec_w2, dec_b2)


# speedup vs baseline: 56.9222x; 1.0075x over previous
import functools

import jax
import jax.numpy as jnp
from jax import lax
from jax.experimental import pallas as pl
from jax.experimental.pallas import tpu as pltpu
from jax.experimental.pallas import tpu_sc as plsc

NC = 2
NS = 16
LL = 16

F32 = jnp.float32
I32 = jnp.int32


def _stage1_body(x_ref, w1_ref, b1_ref, lg_ref, lb_ref, w2_ref, b2_ref,
                 gw_ref, atts_ref, attd_ref, skw_ref, skb_ref,
                 xh_ref, asrc_ref, adst_ref, sself_ref, skip_ref):
    x = x_ref[...]
    h = jnp.dot(x, w1_ref[...], preferred_element_type=F32) + b1_ref[...]
    mu = jnp.mean(h, axis=-1, keepdims=True)
    var = jnp.mean((h - mu) ** 2, axis=-1, keepdims=True)
    h = (h - mu) * lax.rsqrt(var + 1e-5) * lg_ref[...] + lb_ref[...]
    h = jnp.maximum(h, 0.0)
    h = jnp.dot(h, w2_ref[...], preferred_element_type=F32) + b2_ref[...]
    xh = jnp.dot(h, gw_ref[...], preferred_element_type=F32)
    hc = xh.shape[1]
    c = hc // 2
    ps = xh * atts_ref[...]
    pd = xh * attd_ref[...]
    a_s = jnp.stack([jnp.sum(ps[:, :c], axis=1), jnp.sum(ps[:, c:], axis=1)],
                    axis=1)
    a_d = jnp.stack([jnp.sum(pd[:, :c], axis=1), jnp.sum(pd[:, c:], axis=1)],
                    axis=1)
    e_self = a_s + a_d
    e_self = jnp.where(e_self > 0, e_self, 0.2 * e_self)
    sself_ref[...] = jnp.exp(e_self)
    asrc_ref[...] = a_s
    adst_ref[...] = a_d
    skip_ref[...] = jnp.dot(h, skw_ref[...], preferred_element_type=F32) \
        + skb_ref[...]
    for p in range(4):
        xh_ref[p] = xh[:, p * 128:(p + 1) * 128]


def _stage1(x, enc_w1, enc_b1, ln_g, ln_b, enc_w2, enc_b2, gat_W,
            att_src, att_dst, skip_W, skip_b):
    n, d_in = x.shape
    hc = gat_W.shape[1]
    bn = 1000
    grid = (n // bn,)
    full = lambda *shape: pl.BlockSpec(shape, lambda i: (0,) * len(shape))
    row = lambda *shape: pl.BlockSpec(shape, lambda i: (i,) + (0,) * (len(shape) - 1))
    return pl.pallas_call(
        _stage1_body,
        grid=grid,
        in_specs=[
            row(bn, d_in),
            full(d_in, 128), full(1, 128), full(1, 128), full(1, 128),
            full(128, 128), full(1, 128),
            full(128, hc), full(1, hc), full(1, hc),
            full(128, hc), full(1, hc),
        ],
        out_specs=[
            pl.BlockSpec((4, bn, 128), lambda i: (0, i, 0)),
            row(bn, 2), row(bn, 2), row(bn, 2),
            row(bn, hc),
        ],
        out_shape=[
            jax.ShapeDtypeStruct((4, n, 128), F32),
            jax.ShapeDtypeStruct((n, 2), F32),
            jax.ShapeDtypeStruct((n, 2), F32),
            jax.ShapeDtypeStruct((n, 2), F32),
            jax.ShapeDtypeStruct((n, hc), F32),
        ],
    )(x, enc_w1, enc_b1.reshape(1, -1), ln_g.reshape(1, -1),
      ln_b.reshape(1, -1), enc_w2, enc_b2.reshape(1, -1), gat_W,
      att_src.reshape(1, -1), att_dst.reshape(1, -1), skip_W,
      skip_b.reshape(1, -1))


def _splat(vec, lane):
    idx = jnp.full((LL, 1), lane, I32)
    dn = lax.GatherDimensionNumbers(offset_dims=(), collapsed_slice_dims=(0,),
                                    start_index_map=(0,))
    return lax.gather(vec, idx, dn, slice_sizes=(1,),
                      mode=lax.GatherScatterMode.PROMISE_IN_BOUNDS)


def _sc_body(n, nacc, nden, nwin, xh_ref, asrc_ref, adst_ref, srcm_ref,
             dstm_ref, acc_out, den_out,
             src_i, dst_i, is2_v, id2_v, as_v, ad_v, s_v, gbuf, zbuf,
             sem_g, sem_s, sem_d, sem_p, acc_sp, s_sp, den_sp, aS_sp, aD_sp):
    c = lax.axis_index("c")
    t = lax.axis_index("s")
    rpt = 624
    tail0 = rpt * NS
    nw_t = (nwin + NS - 1 - t) // NS

    def _zero_zbuf(i, _):
        zbuf[pl.ds(i * LL, LL)] = jnp.zeros((LL,), F32)
        return 0
    lax.fori_loop(0, zbuf.shape[0] // LL, _zero_zbuf, 0)

    @pl.when(t == 0)
    def _():
        pltpu.sync_copy(asrc_ref, aS_sp)
        pltpu.sync_copy(adst_ref, aD_sp)
        nz = zbuf.shape[0]
        for k in range(nden // nz):
            pltpu.sync_copy(zbuf, den_sp.at[pl.ds(k * nz, nz)])
    plsc.subcore_barrier()

    cvec = jnp.full((LL,), c, I32)

    for q in range(2):
        def _zero_g(i, _):
            j = i // 8
            k = i % 8
            gbuf[j, pl.ds(k * LL, LL)] = jnp.zeros((LL,), F32)
            return 0
        lax.fori_loop(0, 256 * 8, _zero_g, 0)
        pltpu.sync_copy(gbuf, acc_sp.at[pl.ds(t * rpt, 256)])
        pltpu.sync_copy(gbuf, acc_sp.at[pl.ds(t * rpt + 256, 256)])
        pltpu.sync_copy(gbuf.at[pl.ds(0, rpt - 512)],
                        acc_sp.at[pl.ds(t * rpt + 512, rpt - 512)])

        @pl.when(t == NS - 1)
        def _():
            pltpu.sync_copy(gbuf.at[pl.ds(0, nacc - tail0)],
                            acc_sp.at[pl.ds(tail0, nacc - tail0)])
        plsc.subcore_barrier()

        off = (2 * c + q) * n

        def _phase_b(i, _):
            w = t + i * NS
            r0 = w * 8
            ld = [pltpu.async_copy(srcm_ref.at[pl.ds(r0, 8)], src_i, sem_g),
                  pltpu.async_copy(dstm_ref.at[pl.ds(r0, 8)], dst_i, sem_g)]
            if q == 1:
                ld.append(pltpu.async_copy(s_sp.at[c, pl.ds(r0, 8)], s_v,
                                           sem_g))
            for d in ld:
                d.wait()

            def _gath(k, sem=None):
                b = (k % 2) * 128
                return pltpu.async_copy(xh_ref.at[src_i.at[k]],
                                        gbuf.at[pl.ds(b, 128)],
                                        sem_g if sem is None else sem)

            if q == 0:
                def adj0(g, _):
                    j = g // 8
                    k = g % 8
                    sv = src_i[j, pl.ds(k * LL, LL)]
                    dv = dst_i[j, pl.ds(k * LL, LL)]
                    is2_v[j, pl.ds(k * LL, LL)] = sv * 2 + cvec
                    id2_v[j, pl.ds(k * LL, LL)] = dv * 2 + cvec
                    src_i[j, pl.ds(k * LL, LL)] = sv + off
                    return 0
                lax.fori_loop(0, 64, adj0, 0)
                g0 = _gath(0, sem_p)
                ads = [pltpu.async_copy(aS_sp.at[is2_v.at[j]], as_v.at[j],
                                        sem_g) for j in range(8)]
                ads += [pltpu.async_copy(aD_sp.at[id2_v.at[j]], ad_v.at[j],
                                         sem_g) for j in range(8)]
                for d in ads:
                    d.wait()

                def grp(g, _):
                    j = g // 8
                    k = g % 8
                    e = as_v[j, pl.ds(k * LL, LL)] + ad_v[j, pl.ds(k * LL, LL)]
                    e = jnp.where(e > 0, e, 0.2 * e)
                    s_v[j, pl.ds(k * LL, LL)] = jnp.exp(e)
                    return 0
                lax.fori_loop(0, 64, grp, 0)
                pltpu.sync_copy(s_v, s_sp.at[c, pl.ds(r0, 8)])
                dds = [pltpu.async_copy(s_v.at[j], den_sp.at[dst_i.at[j]],
                                        sem_d, add=True) for j in range(8)]
            else:
                def adj(g, _):
                    j = g // 8
                    k = g % 8
                    src_i[j, pl.ds(k * LL, LL)] = \
                        src_i[j, pl.ds(k * LL, LL)] + off
                    return 0
                lax.fori_loop(0, 64, adj, 0)
                dds = []
                g0 = _gath(0, sem_p)

            def _mul(k):
                b = (k % 2) * 128

                def mgrp(g, _):
                    svec = s_v[k, pl.ds(g * LL, LL)]
                    e0 = b + g * LL
                    for jl in range(LL):
                        sj = _splat(svec, jl)
                        ei = e0 + jl
                        for m in range(8):
                            gbuf[ei, pl.ds(m * LL, LL)] = \
                                gbuf[ei, pl.ds(m * LL, LL)] * sj
                    return 0
                lax.fori_loop(0, 8, mgrp, 0)

            def _scat(k):
                b = (k % 2) * 128
                return pltpu.async_copy(gbuf.at[pl.ds(b, 128)],
                                        acc_sp.at[dst_i.at[k]], sem_s,
                                        add=True)

            gds = [None] * 8
            sds = [None] * 8
            gds[0] = g0
            for k in range(8):
                if k + 1 < 8:
                    if k - 1 >= 0:
                        sds[k - 1].wait()
                    gds[k + 1] = _gath(k + 1)
                gds[k].wait()
                _mul(k)
                sds[k] = _scat(k)
            sds[6].wait()
            sds[7].wait()
            for d in dds:
                d.wait()
            return 0
        lax.fori_loop(0, nw_t, _phase_b, 0)
        plsc.subcore_barrier()
        if q == 0:
            @pl.when(t == 0)
            def _():
                pltpu.sync_copy(den_sp, den_out.at[c, 0])
        pltpu.sync_copy(acc_sp.at[pl.ds(t * rpt, rpt)],
                        acc_out.at[c, q, pl.ds(t * rpt, rpt)])

        @pl.when(t == NS - 1)
        def _():
            pltpu.sync_copy(acc_sp.at[pl.ds(tail0, n - tail0)],
                            acc_out.at[c, q, pl.ds(tail0, n - tail0)])
        plsc.subcore_barrier()


def _stage2(xh_flat, a_src, a_dst, srcm, dstm):
    n2 = a_src.shape[0] // 2
    nwin = srcm.shape[0] // 8
    nacc = n2 + 16
    nden = n2 + 2288
    mesh = plsc.VectorSubcoreMesh(core_axis_name="c", subcore_axis_name="s")
    fn = pl.kernel(
        functools.partial(_sc_body, n2, nacc, nden, nwin),
        out_type=(jax.ShapeDtypeStruct((2, 2, n2, 128), F32),
                  jax.ShapeDtypeStruct((2, 1, nden), F32)),
        mesh=mesh,
        scratch_types=[
            pltpu.VMEM((8, 128), I32),
            pltpu.VMEM((8, 128), I32),
            pltpu.VMEM((8, 128), I32),
            pltpu.VMEM((8, 128), I32),
            pltpu.VMEM((8, 128), F32),
            pltpu.VMEM((8, 128), F32),
            pltpu.VMEM((8, 128), F32),
            pltpu.VMEM((256, 128), F32),
            pltpu.VMEM((1024,), F32),
            pltpu.SemaphoreType.DMA,
            pltpu.SemaphoreType.DMA,
            pltpu.SemaphoreType.DMA,
            pltpu.SemaphoreType.DMA,
            pltpu.VMEM_SHARED((nacc, 128), F32),
            pltpu.HBM((2, srcm.shape[0], 128), F32),
            pltpu.VMEM_SHARED((nden,), F32),
            pltpu.VMEM_SHARED((2 * n2,), F32),
            pltpu.VMEM_SHARED((2 * n2,), F32),
        ],
    )
    return fn(xh_flat, a_src, a_dst, srcm, dstm)


def _stage3_body(acc_ref, xh_ref, den_ref, ss_ref, skip_ref, gb_ref,
                 dw1_ref, db1_ref, dw2_ref, db2_ref, out_ref):
    ss = ss_ref[...]
    den = den_ref[...] + ss + 1e-16
    parts = []
    for p in range(4):
        h = p // 2
        num = acc_ref[p] + ss[:, h:h + 1] * xh_ref[p]
        parts.append(num / den[:, h:h + 1])
    conv = jnp.concatenate(parts, axis=1) + gb_ref[...]
    hm = conv + skip_ref[...]
    hm = jnp.where(hm > 0, hm, 0.1 * (jnp.exp(hm) - 1.0))
    d1 = jnp.dot(hm, dw1_ref[...], preferred_element_type=F32) + db1_ref[...]
    d1 = jnp.where(d1 > 0, d1, 0.1 * d1)
    out_ref[...] = jnp.dot(d1, dw2_ref[...], preferred_element_type=F32) \
        + db2_ref[...]


def _stage3(acc, xh_stack, den_t, s_self, skip, gat_b, dec_w1, dec_b1,
            dec_w2, dec_b2):
    n = s_self.shape[0]
    hc = skip.shape[1]
    bn = 1000
    grid = (n // bn,)
    full = lambda *shape: pl.BlockSpec(shape, lambda i: (0,) * len(shape))
    row = lambda *shape: pl.BlockSpec(shape, lambda i: (i,) + (0,) * (len(shape) - 1))
    return pl.pallas_call(
        _stage3_body,
        grid=grid,
        in_specs=[
            pl.BlockSpec((4, bn, 128), lambda i: (0, i, 0)),
            pl.BlockSpec((4, bn, 128), lambda i: (0, i, 0)),
            row(bn, 2), row(bn, 2), row(bn, hc),
            full(1, hc),
            full(hc, 256), full(1, 256), full(256, 128), full(1, 128),
        ],
        out_specs=row(bn, 128),
        out_shape=jax.ShapeDtypeStruct((n, 128), F32),
    )(acc, xh_stack, den_t, s_self, skip, gat_b.reshape(1, -1),
      dec_w1, dec_b1.reshape(1, -1), dec_w2, dec_b2.reshape(1, -1))


def kernel(x, edge_index, enc_w1, enc_b1, ln_g, ln_b, enc_w2, enc_b2,
           gat_W, att_src, att_dst, gat_b, skip_W, skip_b,
           dec_w1, dec_b1, dec_w2, dec_b2):
    n = x.shape[0]
    e = edge_index.shape[1]
    xh_stack, a_src, a_dst, s_self, skip = _stage1(
        x, enc_w1, enc_b1, ln_g, ln_b, enc_w2, enc_b2, gat_W,
        att_src, att_dst, skip_W, skip_b)
    xh_flat = xh_stack.reshape(4 * n, 128)
    epad = (-e) % 1024
    src_pad = jnp.zeros((epad,), I32)
    dst_pad = n + (jnp.arange(epad, dtype=I32) % 16)
    srcm = jnp.concatenate([edge_index[0], src_pad]).reshape(-1, 128)
    dstm = jnp.concatenate([edge_index[1], dst_pad]).reshape(-1, 128)
    acc, den = _stage2(xh_flat, a_src.reshape(-1), a_dst.reshape(-1),
                       srcm, dstm)
    den_t = den[:, 0, :n].T
    return _stage3(acc.reshape(4, n, 128), xh_stack, den_t, s_self, skip,
                   gat_b, dec_w1, dec_b1, dec_w2, dec_b2)


# speedup vs baseline: 57.9718x; 1.0184x over previous
import functools

import jax
import jax.numpy as jnp
from jax import lax
from jax.experimental import pallas as pl
from jax.experimental.pallas import tpu as pltpu
from jax.experimental.pallas import tpu_sc as plsc

NC = 2
NS = 16
LL = 16

F32 = jnp.float32
I32 = jnp.int32


def _stage1_body(x_ref, w1_ref, b1_ref, lg_ref, lb_ref, w2_ref, b2_ref,
                 gw_ref, atts_ref, attd_ref, skw_ref, skb_ref,
                 xh_ref, asrc_ref, adst_ref, sself_ref, skip_ref):
    x = x_ref[...]
    h = jnp.dot(x, w1_ref[...], preferred_element_type=F32) + b1_ref[...]
    mu = jnp.mean(h, axis=-1, keepdims=True)
    var = jnp.mean((h - mu) ** 2, axis=-1, keepdims=True)
    h = (h - mu) * lax.rsqrt(var + 1e-5) * lg_ref[...] + lb_ref[...]
    h = jnp.maximum(h, 0.0)
    h = jnp.dot(h, w2_ref[...], preferred_element_type=F32) + b2_ref[...]
    xh = jnp.dot(h, gw_ref[...], preferred_element_type=F32)
    hc = xh.shape[1]
    c = hc // 2
    ps = xh * atts_ref[...]
    pd = xh * attd_ref[...]
    a_s = jnp.stack([jnp.sum(ps[:, :c], axis=1), jnp.sum(ps[:, c:], axis=1)],
                    axis=1)
    a_d = jnp.stack([jnp.sum(pd[:, :c], axis=1), jnp.sum(pd[:, c:], axis=1)],
                    axis=1)
    e_self = a_s + a_d
    e_self = jnp.where(e_self > 0, e_self, 0.2 * e_self)
    sself_ref[...] = jnp.exp(e_self)
    asrc_ref[...] = a_s
    adst_ref[...] = a_d
    skip_ref[...] = jnp.dot(h, skw_ref[...], preferred_element_type=F32) \
        + skb_ref[...]
    for p in range(4):
        xh_ref[p] = xh[:, p * 128:(p + 1) * 128]


def _stage1(x, enc_w1, enc_b1, ln_g, ln_b, enc_w2, enc_b2, gat_W,
            att_src, att_dst, skip_W, skip_b):
    n, d_in = x.shape
    hc = gat_W.shape[1]
    bn = 1000
    grid = (n // bn,)
    full = lambda *shape: pl.BlockSpec(shape, lambda i: (0,) * len(shape))
    row = lambda *shape: pl.BlockSpec(shape, lambda i: (i,) + (0,) * (len(shape) - 1))
    return pl.pallas_call(
        _stage1_body,
        grid=grid,
        in_specs=[
            row(bn, d_in),
            full(d_in, 128), full(1, 128), full(1, 128), full(1, 128),
            full(128, 128), full(1, 128),
            full(128, hc), full(1, hc), full(1, hc),
            full(128, hc), full(1, hc),
        ],
        out_specs=[
            pl.BlockSpec((4, bn, 128), lambda i: (0, i, 0)),
            row(bn, 2), row(bn, 2), row(bn, 2),
            row(bn, hc),
        ],
        out_shape=[
            jax.ShapeDtypeStruct((4, n, 128), F32),
            jax.ShapeDtypeStruct((n, 2), F32),
            jax.ShapeDtypeStruct((n, 2), F32),
            jax.ShapeDtypeStruct((n, 2), F32),
            jax.ShapeDtypeStruct((n, hc), F32),
        ],
    )(x, enc_w1, enc_b1.reshape(1, -1), ln_g.reshape(1, -1),
      ln_b.reshape(1, -1), enc_w2, enc_b2.reshape(1, -1), gat_W,
      att_src.reshape(1, -1), att_dst.reshape(1, -1), skip_W,
      skip_b.reshape(1, -1))


def _splat(vec, lane):
    idx = jnp.full((LL, 1), lane, I32)
    dn = lax.GatherDimensionNumbers(offset_dims=(), collapsed_slice_dims=(0,),
                                    start_index_map=(0,))
    return lax.gather(vec, idx, dn, slice_sizes=(1,),
                      mode=lax.GatherScatterMode.PROMISE_IN_BOUNDS)


def _sc_body(n, nacc, nden, nwin, xh_ref, asrc_ref, adst_ref, srcm_ref,
             dstm_ref, acc_out, den_out,
             src_i, dst_i, is2_v, id2_v, as_v, ad_v, s_v, gbuf, zbuf,
             sem_g, sem_s, sem_d, sem_p, sem_l,
             acc_sp, s_sp, den_sp, aS_sp, aD_sp):
    c = lax.axis_index("c")
    t = lax.axis_index("s")
    rpt = 624
    tail0 = rpt * NS
    nw_t = (nwin + NS - 1 - t) // NS

    def _zero_zbuf(i, _):
        zbuf[pl.ds(i * LL, LL)] = jnp.zeros((LL,), F32)
        return 0
    lax.fori_loop(0, zbuf.shape[0] // LL, _zero_zbuf, 0)

    @pl.when(t == 0)
    def _():
        pltpu.sync_copy(asrc_ref, aS_sp)
        pltpu.sync_copy(adst_ref, aD_sp)
        nz = zbuf.shape[0]
        for k in range(nden // nz):
            pltpu.sync_copy(zbuf, den_sp.at[pl.ds(k * nz, nz)])
    plsc.subcore_barrier()

    cvec = jnp.full((LL,), c, I32)

    for q in range(2):
        def _zero_g(i, _):
            j = i // 8
            k = i % 8
            gbuf[j, pl.ds(k * LL, LL)] = jnp.zeros((LL,), F32)
            return 0
        lax.fori_loop(0, 256 * 8, _zero_g, 0)
        pltpu.sync_copy(gbuf, acc_sp.at[pl.ds(t * rpt, 256)])
        pltpu.sync_copy(gbuf, acc_sp.at[pl.ds(t * rpt + 256, 256)])
        pltpu.sync_copy(gbuf.at[pl.ds(0, rpt - 512)],
                        acc_sp.at[pl.ds(t * rpt + 512, rpt - 512)])

        @pl.when(t == NS - 1)
        def _():
            pltpu.sync_copy(gbuf.at[pl.ds(0, nacc - tail0)],
                            acc_sp.at[pl.ds(tail0, nacc - tail0)])
        plsc.subcore_barrier()

        off = (2 * c + q) * n

        def _load_win(i2):
            r2 = (t + i2 * NS) * 8
            b2 = i2 % 2
            pltpu.async_copy(srcm_ref.at[pl.ds(r2, 8)], src_i.at[b2], sem_l)
            pltpu.async_copy(dstm_ref.at[pl.ds(r2, 8)], dst_i.at[b2], sem_l)
            if q == 1:
                pltpu.async_copy(s_sp.at[c, pl.ds(r2, 8)], s_v.at[b2], sem_l)

        @pl.when(nw_t > 0)
        def _():
            _load_win(0)

        def _phase_b(i, _):
            b = i % 2
            w = t + i * NS
            r0 = w * 8
            pltpu.make_async_copy(srcm_ref.at[pl.ds(r0, 8)], src_i.at[b],
                                  sem_l).wait()
            pltpu.make_async_copy(dstm_ref.at[pl.ds(r0, 8)], dst_i.at[b],
                                  sem_l).wait()
            if q == 1:
                pltpu.make_async_copy(s_sp.at[c, pl.ds(r0, 8)], s_v.at[b],
                                      sem_l).wait()

            def _gath(k, sem=None):
                bb = (k % 2) * 128
                return pltpu.async_copy(xh_ref.at[src_i.at[b, k]],
                                        gbuf.at[pl.ds(bb, 128)],
                                        sem_g if sem is None else sem)

            if q == 0:
                def adj0(g, _):
                    j = g // 8
                    k = g % 8
                    sv = src_i[b, j, pl.ds(k * LL, LL)]
                    dv = dst_i[b, j, pl.ds(k * LL, LL)]
                    is2_v[j, pl.ds(k * LL, LL)] = sv * 2 + cvec
                    id2_v[j, pl.ds(k * LL, LL)] = dv * 2 + cvec
                    src_i[b, j, pl.ds(k * LL, LL)] = sv + off
                    return 0
                lax.fori_loop(0, 64, adj0, 0)
                g0 = _gath(0, sem_p)

                @pl.when(i + 1 < nw_t)
                def _():
                    _load_win(i + 1)
                ads = [pltpu.async_copy(aS_sp.at[is2_v.at[j]], as_v.at[j],
                                        sem_g) for j in range(8)]
                ads += [pltpu.async_copy(aD_sp.at[id2_v.at[j]], ad_v.at[j],
                                         sem_g) for j in range(8)]
                for d in ads:
                    d.wait()

                def grp(g, _):
                    j = g // 8
                    k = g % 8
                    e = as_v[j, pl.ds(k * LL, LL)] + ad_v[j, pl.ds(k * LL, LL)]
                    e = jnp.where(e > 0, e, 0.2 * e)
                    s_v[b, j, pl.ds(k * LL, LL)] = jnp.exp(e)
                    return 0
                lax.fori_loop(0, 64, grp, 0)
                pltpu.sync_copy(s_v.at[b], s_sp.at[c, pl.ds(r0, 8)])
                dds = [pltpu.async_copy(s_v.at[b, j],
                                        den_sp.at[dst_i.at[b, j]],
                                        sem_d, add=True) for j in range(8)]
            else:
                def adj(g, _):
                    j = g // 8
                    k = g % 8
                    src_i[b, j, pl.ds(k * LL, LL)] = \
                        src_i[b, j, pl.ds(k * LL, LL)] + off
                    return 0
                lax.fori_loop(0, 64, adj, 0)
                dds = []
                g0 = _gath(0, sem_p)

                @pl.when(i + 1 < nw_t)
                def _():
                    _load_win(i + 1)

            def _mul(k):
                bb = (k % 2) * 128

                def mgrp(g, _):
                    svec = s_v[b, k, pl.ds(g * LL, LL)]
                    e0 = bb + g * LL
                    for jl in range(LL):
                        sj = _splat(svec, jl)
                        ei = e0 + jl
                        for m in range(8):
                            gbuf[ei, pl.ds(m * LL, LL)] = \
                                gbuf[ei, pl.ds(m * LL, LL)] * sj
                    return 0
                lax.fori_loop(0, 8, mgrp, 0)

            def _scat(k):
                bb = (k % 2) * 128
                return pltpu.async_copy(gbuf.at[pl.ds(bb, 128)],
                                        acc_sp.at[dst_i.at[b, k]], sem_s,
                                        add=True)

            gds = [None] * 8
            sds = [None] * 8
            gds[0] = g0
            for k in range(8):
                if k + 1 < 8:
                    if k - 1 >= 0:
                        sds[k - 1].wait()
                    gds[k + 1] = _gath(k + 1)
                gds[k].wait()
                _mul(k)
                sds[k] = _scat(k)
            sds[6].wait()
            sds[7].wait()
            for d in dds:
                d.wait()
            return 0
        lax.fori_loop(0, nw_t, _phase_b, 0)
        plsc.subcore_barrier()
        if q == 0:
            @pl.when(t == 0)
            def _():
                pltpu.sync_copy(den_sp, den_out.at[c, 0])
        pltpu.sync_copy(acc_sp.at[pl.ds(t * rpt, rpt)],
                        acc_out.at[c, q, pl.ds(t * rpt, rpt)])

        @pl.when(t == NS - 1)
        def _():
            pltpu.sync_copy(acc_sp.at[pl.ds(tail0, n - tail0)],
                            acc_out.at[c, q, pl.ds(tail0, n - tail0)])
        plsc.subcore_barrier()


def _stage2(xh_flat, a_src, a_dst, srcm, dstm):
    n2 = a_src.shape[0] // 2
    nwin = srcm.shape[0] // 8
    nacc = n2 + 16
    nden = n2 + 2288
    mesh = plsc.VectorSubcoreMesh(core_axis_name="c", subcore_axis_name="s")
    fn = pl.kernel(
        functools.partial(_sc_body, n2, nacc, nden, nwin),
        out_type=(jax.ShapeDtypeStruct((2, 2, n2, 128), F32),
                  jax.ShapeDtypeStruct((2, 1, nden), F32)),
        mesh=mesh,
        scratch_types=[
            pltpu.VMEM((2, 8, 128), I32),
            pltpu.VMEM((2, 8, 128), I32),
            pltpu.VMEM((8, 128), I32),
            pltpu.VMEM((8, 128), I32),
            pltpu.VMEM((8, 128), F32),
            pltpu.VMEM((8, 128), F32),
            pltpu.VMEM((2, 8, 128), F32),
            pltpu.VMEM((256, 128), F32),
            pltpu.VMEM((1024,), F32),
            pltpu.SemaphoreType.DMA,
            pltpu.SemaphoreType.DMA,
            pltpu.SemaphoreType.DMA,
            pltpu.SemaphoreType.DMA,
            pltpu.SemaphoreType.DMA,
            pltpu.VMEM_SHARED((nacc, 128), F32),
            pltpu.HBM((2, srcm.shape[0], 128), F32),
            pltpu.VMEM_SHARED((nden,), F32),
            pltpu.VMEM_SHARED((2 * n2,), F32),
            pltpu.VMEM_SHARED((2 * n2,), F32),
        ],
    )
    return fn(xh_flat, a_src, a_dst, srcm, dstm)


def _stage3_body(acc_ref, xh_ref, den_ref, ss_ref, skip_ref, gb_ref,
                 dw1_ref, db1_ref, dw2_ref, db2_ref, out_ref):
    ss = ss_ref[...]
    den = den_ref[...] + ss + 1e-16
    parts = []
    for p in range(4):
        h = p // 2
        num = acc_ref[p] + ss[:, h:h + 1] * xh_ref[p]
        parts.append(num / den[:, h:h + 1])
    conv = jnp.concatenate(parts, axis=1) + gb_ref[...]
    hm = conv + skip_ref[...]
    hm = jnp.where(hm > 0, hm, 0.1 * (jnp.exp(hm) - 1.0))
    d1 = jnp.dot(hm, dw1_ref[...], preferred_element_type=F32) + db1_ref[...]
    d1 = jnp.where(d1 > 0, d1, 0.1 * d1)
    out_ref[...] = jnp.dot(d1, dw2_ref[...], preferred_element_type=F32) \
        + db2_ref[...]


def _stage3(acc, xh_stack, den_t, s_self, skip, gat_b, dec_w1, dec_b1,
            dec_w2, dec_b2):
    n = s_self.shape[0]
    hc = skip.shape[1]
    bn = 1000
    grid = (n // bn,)
    full = lambda *shape: pl.BlockSpec(shape, lambda i: (0,) * len(shape))
    row = lambda *shape: pl.BlockSpec(shape, lambda i: (i,) + (0,) * (len(shape) - 1))
    return pl.pallas_call(
        _stage3_body,
        grid=grid,
        in_specs=[
            pl.BlockSpec((4, bn, 128), lambda i: (0, i, 0)),
            pl.BlockSpec((4, bn, 128), lambda i: (0, i, 0)),
            row(bn, 2), row(bn, 2), row(bn, hc),
            full(1, hc),
            full(hc, 256), full(1, 256), full(256, 128), full(1, 128),
        ],
        out_specs=row(bn, 128),
        out_shape=jax.ShapeDtypeStruct((n, 128), F32),
    )(acc, xh_stack, den_t, s_self, skip, gat_b.reshape(1, -1),
      dec_w1, dec_b1.reshape(1, -1), dec_w2, dec_b2.reshape(1, -1))


def kernel(x, edge_index, enc_w1, enc_b1, ln_g, ln_b, enc_w2, enc_b2,
           gat_W, att_src, att_dst, gat_b, skip_W, skip_b,
           dec_w1, dec_b1, dec_w2, dec_b2):
    n = x.shape[0]
    e = edge_index.shape[1]
    xh_stack, a_src, a_dst, s_self, skip = _stage1(
        x, enc_w1, enc_b1, ln_g, ln_b, enc_w2, enc_b2, gat_W,
        att_src, att_dst, skip_W, skip_b)
    xh_flat = xh_stack.reshape(4 * n, 128)
    epad = (-e) % 1024
    src_pad = jnp.zeros((epad,), I32)
    dst_pad = n + (jnp.arange(epad, dtype=I32) % 16)
    srcm = jnp.concatenate([edge_index[0], src_pad]).reshape(-1, 128)
    dstm = jnp.concatenate([edge_index[1], dst_pad]).reshape(-1, 128)
    acc, den = _stage2(xh_flat, a_src.reshape(-1), a_dst.reshape(-1),
                       srcm, dstm)
    den_t = den[:, 0, :n].T
    return _stage3(acc.reshape(4, n, 128), xh_stack, den_t, s_self, skip,
                   gat_b, dec_w1, dec_b1, dec_w2, dec_b2)
